# baseline (speedup 1.0000x reference)
import functools

import jax
import jax.numpy as jnp
from jax import lax
from jax.experimental import pallas as pl
from jax.experimental.pallas import tpu as pltpu
from jax.experimental.pallas import tpu_sc as plsc

N = 10000
E = 320000
D = 128

NP = 10240
C = 96
NCH0 = 60
NCH1 = 156
NCHT = NCH0 + NCH1
EPTOT = 16 * NCHT * C
RPT = NP // 16


def _sc_spmm(h, srcp, dstp, wp, zrows):
    mesh = plsc.VectorSubcoreMesh(core_axis_name="c", subcore_axis_name="s")

    @functools.partial(
        pl.kernel,
        out_type=jax.ShapeDtypeStruct((2, NP, D), jnp.float32),
        mesh=mesh,
        scratch_types=[
            pltpu.VMEM_SHARED((NP, D), jnp.float32),
            pltpu.VMEM((3, 1, C), jnp.int32),
            pltpu.VMEM((3, 1, C), jnp.int32),
            pltpu.VMEM((3, 1, C), jnp.float32),
            pltpu.VMEM((3, C, D), jnp.float32),
            pltpu.SemaphoreType.DMA((3,)),
            pltpu.SemaphoreType.DMA((3,)),
            pltpu.SemaphoreType.DMA,
        ],
    )
    def k(h_hbm, src_hbm, dst2_hbm, w_hbm, z_hbm, out_hbm,
          acc, siv, div, wv, rows3, isems, gsems, ssem):
        c = lax.axis_index("c")
        s = lax.axis_index("s")
        r0 = s * RPT
        nch = jnp.where(c == 0, NCH0, NCH1)
        cbase = s * NCHT + c * NCH0
        base = cbase * C

        def eload(jj, b, wait):
            args = (
                (src_hbm.at[pl.ds(base + jj * C, C)], siv.at[b, 0], isems.at[b]),
                (dst2_hbm.at[cbase + jj], div.at[b], isems.at[b]),
                (w_hbm.at[pl.ds(base + jj * C, C)], wv.at[b, 0], isems.at[b]),
            )
            for a in args:
                if wait:
                    pltpu.make_async_copy(*a).wait()
                else:
                    pltpu.async_copy(*a)

        def gather(jj, b, wait):
            a = (h_hbm.at[siv.at[b, 0]], rows3.at[b], gsems.at[b])
            if wait:
                pltpu.make_async_copy(*a).wait()
            else:
                pltpu.async_copy(*a)

        def scat_wait(b):
            pltpu.make_async_copy(rows3.at[b], acc.at[div.at[b, 0]], ssem).wait()

        eload(0, 0, False)
        eload(1, 1, False)
        pltpu.sync_copy(z_hbm, acc.at[pl.ds(r0, RPT)])
        eload(0, 0, True)
        gather(0, 0, False)
        plsc.subcore_barrier()

        def outer(j, carry):
            for u in range(3):
                jj = 3 * j + u
                b0, b1, b2 = u, (u + 1) % 3, (u + 2) % 3

                @pl.when(jj + 1 < nch)
                def _():
                    eload(jj + 1, b1, True)
                    gather(jj + 1, b1, False)

                @pl.when(jj >= 1)
                def _():
                    scat_wait(b2)

                @pl.when(jj + 2 < nch)
                def _():
                    eload(jj + 2, b2, False)

                gather(jj, b0, True)

                def group(g, cc):
                    wg = wv[b0, 0, pl.ds(g * 16, 16)]
                    for i in range(16):
                        ww = wg[i]
                        e = g * 16 + i
                        for t in range(D // 16):
                            sl = pl.ds(t * 16, 16)
                            rows3[b0, e, sl] = rows3[b0, e, sl] * ww
                    return cc

                lax.fori_loop(0, C // 16, group, 0)
                pltpu.async_copy(rows3.at[b0], acc.at[div.at[b0, 0]], ssem, add=True)
            return carry

        lax.fori_loop(0, nch // 3, outer, 0)
        scat_wait(2)
        plsc.subcore_barrier()
        pltpu.sync_copy(acc.at[pl.ds(r0, RPT)], out_hbm.at[c, pl.ds(r0, RPT)])

    return k(h, srcp, dstp, wp, zrows)


def _combine_body(p_ref, o_ref):
    o_ref[...] = p_ref[0] + p_ref[1]


def _combine(p):
    blk = 2048
    return pl.pallas_call(
        _combine_body,
        grid=(NP // blk,),
        in_specs=[pl.BlockSpec((2, blk, D), lambda i: (0, i, 0))],
        out_specs=pl.BlockSpec((blk, D), lambda i: (i, 0)),
        out_shape=jax.ShapeDtypeStruct((NP, D), jnp.float32),
    )(p)


def _proj_body(h_ref, f1_ref, f2_ref, wt_ref, b_ref, o_ref, *, act):
    z = (
        jnp.dot(h_ref[...], wt_ref[0:D], preferred_element_type=jnp.float32)
        + jnp.dot(f1_ref[...], wt_ref[D:2 * D], preferred_element_type=jnp.float32)
        + jnp.dot(f2_ref[...], wt_ref[2 * D:3 * D], preferred_element_type=jnp.float32)
        + b_ref[...]
    )
    if act:
        z = jnp.where(z >= 0, z, 0.01 * z)
    o_ref[...] = z


def _proj(h, f1, f2, W, b, act, out_rows, blk):
    wt = W.T
    out_d = W.shape[0]
    rspec = pl.BlockSpec((blk, D), lambda i: (i, 0))
    return pl.pallas_call(
        functools.partial(_proj_body, act=act),
        grid=(out_rows // blk,),
        in_specs=[
            rspec, rspec, rspec,
            pl.BlockSpec((3 * D, out_d), lambda i: (0, 0)),
            pl.BlockSpec((out_d,), lambda i: (0,)),
        ],
        out_specs=pl.BlockSpec((blk, out_d), lambda i: (i, 0)),
        out_shape=jax.ShapeDtypeStruct((out_rows, out_d), jnp.float32),
    )(h, f1, f2, wt, b)


def kernel(x, edge_index, edge_weight, W0, b0, W1, b1):
    dst = edge_index[0]
    src = edge_index[1]
    pad = EPTOT - E
    srcp = jnp.pad(src.astype(jnp.int32), (0, pad))
    dstp = jnp.pad(dst.astype(jnp.int32), (0, pad)).reshape(16 * NCHT, 1, C)
    wp = jnp.pad(edge_weight, (0, pad))
    zrows = jnp.zeros((RPT, D), jnp.float32)

    f1 = _combine(_sc_spmm(x, srcp, dstp, wp, zrows))
    f2 = _combine(_sc_spmm(f1, srcp, dstp, wp, zrows))
    h1 = _proj(x, f1, f2, W0, b0, True, NP, 2048)
    g1 = _combine(_sc_spmm(h1, srcp, dstp, wp, zrows))
    g2 = _combine(_sc_spmm(g1, srcp, dstp, wp, zrows))
    return _proj(h1, g1, g2, W1, b1, False, N, 1000)


# speedup vs baseline: 1.3193x; 1.3193x over previous
import functools

import jax
import jax.numpy as jnp
from jax import lax
from jax.experimental import pallas as pl
from jax.experimental.pallas import tpu as pltpu
from jax.experimental.pallas import tpu_sc as plsc

N = 10000
E = 320000
D = 128

NP = 10240
C = 96
NCH0 = 156
NCH1 = 60
NCHT = NCH0 + NCH1
EPTOT = 16 * NCHT * C
RPT = NP // 16


def _sc_spmm(h, srcp, dstp, wp, zrows):
    mesh = plsc.VectorSubcoreMesh(core_axis_name="c", subcore_axis_name="s")

    @functools.partial(
        pl.kernel,
        out_type=jax.ShapeDtypeStruct((2, NP, D), jnp.float32),
        mesh=mesh,
        scratch_types=[
            pltpu.VMEM_SHARED((NP, D), jnp.float32),
            pltpu.VMEM((3, 1, C), jnp.int32),
            pltpu.VMEM((3, 1, C), jnp.int32),
            pltpu.VMEM((3, 1, C), jnp.float32),
            pltpu.VMEM((3, C, D), jnp.float32),
            pltpu.SemaphoreType.DMA((3,)),
            pltpu.SemaphoreType.DMA((3,)),
            pltpu.SemaphoreType.DMA,
        ],
    )
    def k(h_hbm, src_hbm, dst2_hbm, w_hbm, z_hbm, out_hbm,
          acc, siv, div, wv, rows3, isems, gsems, ssem):
        c = lax.axis_index("c")
        s = lax.axis_index("s")
        r0 = s * RPT
        nch = jnp.where(c == 0, NCH0, NCH1)
        cbase = s * NCHT + c * NCH0
        base = cbase * C

        def eload(jj, b, wait):
            args = (
                (src_hbm.at[pl.ds(base + jj * C, C)], siv.at[b, 0], isems.at[b]),
                (dst2_hbm.at[cbase + jj], div.at[b], isems.at[b]),
                (w_hbm.at[pl.ds(base + jj * C, C)], wv.at[b, 0], isems.at[b]),
            )
            for a in args:
                if wait:
                    pltpu.make_async_copy(*a).wait()
                else:
                    pltpu.async_copy(*a)

        def gather(jj, b, wait):
            a = (h_hbm.at[siv.at[b, 0]], rows3.at[b], gsems.at[b])
            if wait:
                pltpu.make_async_copy(*a).wait()
            else:
                pltpu.async_copy(*a)

        def scat_wait(b):
            pltpu.make_async_copy(rows3.at[b], acc.at[div.at[b, 0]], ssem).wait()

        eload(0, 0, False)
        eload(1, 1, False)
        pltpu.sync_copy(z_hbm, acc.at[pl.ds(r0, RPT)])
        eload(0, 0, True)
        gather(0, 0, False)
        plsc.subcore_barrier()

        def outer(j, carry):
            for u in range(3):
                jj = 3 * j + u
                b0, b1, b2 = u, (u + 1) % 3, (u + 2) % 3

                @pl.when(jj + 1 < nch)
                def _():
                    eload(jj + 1, b1, True)
                    gather(jj + 1, b1, False)

                @pl.when(jj >= 1)
                def _():
                    scat_wait(b2)

                @pl.when(jj + 2 < nch)
                def _():
                    eload(jj + 2, b2, False)

                gather(jj, b0, True)

                def group(g, cc):
                    wg = wv[b0, 0, pl.ds(g * 16, 16)]
                    for i in range(16):
                        ww = wg[i]
                        e = g * 16 + i
                        for t in range(D // 16):
                            sl = pl.ds(t * 16, 16)
                            rows3[b0, e, sl] = rows3[b0, e, sl] * ww
                    return cc

                lax.fori_loop(0, C // 16, group, 0)
                pltpu.async_copy(rows3.at[b0], acc.at[div.at[b0, 0]], ssem, add=True)
            return carry

        lax.fori_loop(0, nch // 3, outer, 0)
        scat_wait(2)
        plsc.subcore_barrier()
        pltpu.sync_copy(acc.at[pl.ds(r0, RPT)], out_hbm.at[c, pl.ds(r0, RPT)])

    return k(h, srcp, dstp, wp, zrows)


def _combine_body(p_ref, o_ref):
    o_ref[...] = p_ref[0] + p_ref[1]


def _combine(p):
    blk = 2048
    return pl.pallas_call(
        _combine_body,
        grid=(NP // blk,),
        in_specs=[pl.BlockSpec((2, blk, D), lambda i: (0, i, 0))],
        out_specs=pl.BlockSpec((blk, D), lambda i: (i, 0)),
        out_shape=jax.ShapeDtypeStruct((NP, D), jnp.float32),
    )(p)


def _proj_body(h_ref, f1_ref, f2_ref, wt_ref, b_ref, o_ref, *, act):
    z = (
        jnp.dot(h_ref[...], wt_ref[0:D], preferred_element_type=jnp.float32)
        + jnp.dot(f1_ref[...], wt_ref[D:2 * D], preferred_element_type=jnp.float32)
        + jnp.dot(f2_ref[...], wt_ref[2 * D:3 * D], preferred_element_type=jnp.float32)
        + b_ref[...]
    )
    if act:
        z = jnp.where(z >= 0, z, 0.01 * z)
    o_ref[...] = z


def _proj(h, f1, f2, W, b, act, out_rows, blk):
    wt = W.T
    out_d = W.shape[0]
    rspec = pl.BlockSpec((blk, D), lambda i: (i, 0))
    return pl.pallas_call(
        functools.partial(_proj_body, act=act),
        grid=(out_rows // blk,),
        in_specs=[
            rspec, rspec, rspec,
            pl.BlockSpec((3 * D, out_d), lambda i: (0, 0)),
            pl.BlockSpec((out_d,), lambda i: (0,)),
        ],
        out_specs=pl.BlockSpec((blk, out_d), lambda i: (i, 0)),
        out_shape=jax.ShapeDtypeStruct((out_rows, out_d), jnp.float32),
    )(h, f1, f2, wt, b)


def kernel(x, edge_index, edge_weight, W0, b0, W1, b1):
    dst = edge_index[0]
    src = edge_index[1]
    pad = EPTOT - E
    srcp = jnp.pad(src.astype(jnp.int32), (0, pad))
    dstp = jnp.pad(dst.astype(jnp.int32), (0, pad)).reshape(16 * NCHT, 1, C)
    wp = jnp.pad(edge_weight, (0, pad))
    zrows = jnp.zeros((RPT, D), jnp.float32)

    f1 = _combine(_sc_spmm(x, srcp, dstp, wp, zrows))
    f2 = _combine(_sc_spmm(f1, srcp, dstp, wp, zrows))
    h1 = _proj(x, f1, f2, W0, b0, True, NP, 2048)
    g1 = _combine(_sc_spmm(h1, srcp, dstp, wp, zrows))
    g2 = _combine(_sc_spmm(g1, srcp, dstp, wp, zrows))
    return _proj(h1, g1, g2, W1, b1, False, N, 1000)


# speedup vs baseline: 1.3945x; 1.0570x over previous
import functools

import jax
import jax.numpy as jnp
from jax import lax
from jax.experimental import pallas as pl
from jax.experimental.pallas import tpu as pltpu
from jax.experimental.pallas import tpu_sc as plsc

N = 10000
E = 320000
D = 128

NP = 10240
C = 96
NCH0 = 147
NCH1 = 69
NCHT = NCH0 + NCH1
EPTOT = 16 * NCHT * C
RPT = NP // 16


def _sc_spmm(h, srcp, dstp, wp):
    mesh = plsc.VectorSubcoreMesh(core_axis_name="c", subcore_axis_name="s")

    @functools.partial(
        pl.kernel,
        out_type=jax.ShapeDtypeStruct((2, NP, D), jnp.float32),
        mesh=mesh,
        scratch_types=[
            pltpu.VMEM_SHARED((NP, D), jnp.float32),
            pltpu.VMEM((3, 1, C), jnp.int32),
            pltpu.VMEM((3, 1, C), jnp.int32),
            pltpu.VMEM((3, 1, C), jnp.float32),
            pltpu.VMEM((3, C, D), jnp.float32),
            pltpu.SemaphoreType.DMA((3,)),
            pltpu.SemaphoreType.DMA((3,)),
            pltpu.SemaphoreType.DMA,
        ],
    )
    def k(h_hbm, src_hbm, dst2_hbm, w_hbm, out_hbm,
          acc, siv, div, wv, rows3, isems, gsems, ssem):
        c = lax.axis_index("c")
        s = lax.axis_index("s")
        r0 = s * RPT
        nch = jnp.where(c == 0, NCH0, NCH1)
        cbase = s * NCHT + c * NCH0
        base = cbase * C

        def eload(jj, b, wait):
            args = (
                (src_hbm.at[pl.ds(base + jj * C, C)], siv.at[b, 0], isems.at[b]),
                (dst2_hbm.at[cbase + jj], div.at[b], isems.at[b]),
                (w_hbm.at[pl.ds(base + jj * C, C)], wv.at[b, 0], isems.at[b]),
            )
            for a in args:
                if wait:
                    pltpu.make_async_copy(*a).wait()
                else:
                    pltpu.async_copy(*a)

        def gather(jj, b, wait):
            a = (h_hbm.at[siv.at[b, 0]], rows3.at[b], gsems.at[b])
            if wait:
                pltpu.make_async_copy(*a).wait()
            else:
                pltpu.async_copy(*a)

        def scat_wait(b):
            pltpu.make_async_copy(rows3.at[b], acc.at[div.at[b, 0]], ssem).wait()

        eload(0, 0, False)
        eload(1, 1, False)

        def zrow(i, cc):
            for t in range(D // 16):
                rows3[0, i, pl.ds(t * 16, 16)] = jnp.zeros((16,), jnp.float32)
            return cc

        lax.fori_loop(0, C, zrow, 0)
        for q in range(RPT // C):
            pltpu.sync_copy(rows3.at[0], acc.at[pl.ds(r0 + q * C, C)])
        pltpu.sync_copy(rows3.at[0, pl.ds(0, RPT % C)],
                        acc.at[pl.ds(r0 + (RPT // C) * C, RPT % C)])

        eload(0, 0, True)
        gather(0, 0, False)
        plsc.subcore_barrier()

        def outer(j, carry):
            for u in range(3):
                jj = 3 * j + u
                b0, b1, b2 = u, (u + 1) % 3, (u + 2) % 3

                @pl.when(jj + 1 < nch)
                def _():
                    eload(jj + 1, b1, True)
                    gather(jj + 1, b1, False)

                @pl.when(jj >= 1)
                def _():
                    scat_wait(b2)

                @pl.when(jj + 2 < nch)
                def _():
                    eload(jj + 2, b2, False)

                gather(jj, b0, True)

                def group(g, cc):
                    wg = wv[b0, 0, pl.ds(g * 16, 16)]
                    for i in range(16):
                        ww = wg[i]
                        e = g * 16 + i
                        for t in range(D // 16):
                            sl = pl.ds(t * 16, 16)
                            rows3[b0, e, sl] = rows3[b0, e, sl] * ww
                    return cc

                lax.fori_loop(0, C // 16, group, 0)
                pltpu.async_copy(rows3.at[b0], acc.at[div.at[b0, 0]], ssem, add=True)
            return carry

        lax.fori_loop(0, nch // 3, outer, 0)
        scat_wait(2)
        plsc.subcore_barrier()
        pltpu.sync_copy(acc.at[pl.ds(r0, RPT)], out_hbm.at[c, pl.ds(r0, RPT)])

    return k(h, srcp, dstp, wp)


def _combine_body(p_ref, o_ref):
    o_ref[...] = p_ref[0] + p_ref[1]


def _combine(p):
    blk = 2048
    return pl.pallas_call(
        _combine_body,
        grid=(NP // blk,),
        in_specs=[pl.BlockSpec((2, blk, D), lambda i: (0, i, 0))],
        out_specs=pl.BlockSpec((blk, D), lambda i: (i, 0)),
        out_shape=jax.ShapeDtypeStruct((NP, D), jnp.float32),
    )(p)


def _proj_body(h_ref, f1_ref, p2_ref, wt_ref, b_ref, o_ref, *, act):
    f2 = p2_ref[0] + p2_ref[1]
    z = (
        jnp.dot(h_ref[...], wt_ref[0:D], preferred_element_type=jnp.float32)
        + jnp.dot(f1_ref[...], wt_ref[D:2 * D], preferred_element_type=jnp.float32)
        + jnp.dot(f2, wt_ref[2 * D:3 * D], preferred_element_type=jnp.float32)
        + b_ref[...]
    )
    if act:
        z = jnp.where(z >= 0, z, 0.01 * z)
    o_ref[...] = z


def _proj(h, f1, p2, W, b, act, out_rows, blk):
    wt = W.T
    out_d = W.shape[0]
    rspec = pl.BlockSpec((blk, D), lambda i: (i, 0))
    return pl.pallas_call(
        functools.partial(_proj_body, act=act),
        grid=(out_rows // blk,),
        in_specs=[
            rspec, rspec,
            pl.BlockSpec((2, blk, D), lambda i: (0, i, 0)),
            pl.BlockSpec((3 * D, out_d), lambda i: (0, 0)),
            pl.BlockSpec((out_d,), lambda i: (0,)),
        ],
        out_specs=pl.BlockSpec((blk, out_d), lambda i: (i, 0)),
        out_shape=jax.ShapeDtypeStruct((out_rows, out_d), jnp.float32),
    )(h, f1, p2, wt, b)


def kernel(x, edge_index, edge_weight, W0, b0, W1, b1):
    dst = edge_index[0]
    src = edge_index[1]
    pad = EPTOT - E
    srcp = jnp.pad(src.astype(jnp.int32), (0, pad))
    dstp = jnp.pad(dst.astype(jnp.int32), (0, pad)).reshape(16 * NCHT, 1, C)
    wp = jnp.pad(edge_weight, (0, pad))

    f1 = _combine(_sc_spmm(x, srcp, dstp, wp))
    p2 = _sc_spmm(f1, srcp, dstp, wp)
    h1 = _proj(x, f1, p2, W0, b0, True, NP, 2048)
    g1 = _combine(_sc_spmm(h1, srcp, dstp, wp))
    pg2 = _sc_spmm(g1, srcp, dstp, wp)
    return _proj(h1, g1, pg2, W1, b1, False, N, 1000)


# speedup vs baseline: 3.5431x; 2.5407x over previous
import functools

import jax
import jax.numpy as jnp
from jax import lax
from jax.experimental import pallas as pl
from jax.experimental.pallas import tpu as pltpu
from jax.experimental.pallas import tpu_sc as plsc

N = 10000
E = 320000
D = 128

NP = 10240
C = 112
NCH0 = 123
NCH1 = 57
NCHT = NCH0 + NCH1
EPTOT = 16 * NCHT * C
RPT = NP // 16


def _sc_spmm(h, srcp, dstp, wp):
    mesh = plsc.VectorSubcoreMesh(core_axis_name="c", subcore_axis_name="s")

    @functools.partial(
        pl.kernel,
        out_type=jax.ShapeDtypeStruct((2, NP, D), jnp.float32),
        mesh=mesh,
        scratch_types=[
            pltpu.VMEM_SHARED((NP, D), jnp.float32),
            pltpu.VMEM((3, 1, C), jnp.int32),
            pltpu.VMEM((3, 1, C), jnp.int32),
            pltpu.VMEM((3, 1, C), jnp.float32),
            pltpu.VMEM((3, C, D), jnp.float32),
            pltpu.SemaphoreType.DMA((3,)),
            pltpu.SemaphoreType.DMA((3,)),
            pltpu.SemaphoreType.DMA,
        ],
    )
    def k(h_hbm, src_hbm, dst2_hbm, w_hbm, out_hbm,
          acc, siv, div, wv, rows3, isems, gsems, ssem):
        c = lax.axis_index("c")
        s = lax.axis_index("s")
        r0 = s * RPT
        nch = jnp.where(c == 0, NCH0, NCH1)
        cbase = s * NCHT + c * NCH0
        base = cbase * C

        def eload(jj, b, wait):
            args = (
                (src_hbm.at[pl.ds(base + jj * C, C)], siv.at[b, 0], isems.at[b]),
                (dst2_hbm.at[cbase + jj], div.at[b], isems.at[b]),
                (w_hbm.at[pl.ds(base + jj * C, C)], wv.at[b, 0], isems.at[b]),
            )
            for a in args:
                if wait:
                    pltpu.make_async_copy(*a).wait()
                else:
                    pltpu.async_copy(*a)

        def gather(jj, b, wait):
            a = (h_hbm.at[siv.at[b, 0]], rows3.at[b], gsems.at[b])
            if wait:
                pltpu.make_async_copy(*a).wait()
            else:
                pltpu.async_copy(*a)

        def scat_wait(b):
            pltpu.make_async_copy(rows3.at[b], acc.at[div.at[b, 0]], ssem).wait()

        eload(0, 0, False)
        eload(1, 1, False)

        def zrow(i, cc):
            for t in range(D // 16):
                rows3[0, i, pl.ds(t * 16, 16)] = jnp.zeros((16,), jnp.float32)
            return cc

        lax.fori_loop(0, C, zrow, 0)
        for q in range(RPT // C):
            pltpu.sync_copy(rows3.at[0], acc.at[pl.ds(r0 + q * C, C)])
        pltpu.sync_copy(rows3.at[0, pl.ds(0, RPT % C)],
                        acc.at[pl.ds(r0 + (RPT // C) * C, RPT % C)])

        eload(0, 0, True)
        gather(0, 0, False)
        plsc.subcore_barrier()

        def outer(j, carry):
            for u in range(3):
                jj = 3 * j + u
                b0, b1, b2 = u, (u + 1) % 3, (u + 2) % 3

                @pl.when(jj + 1 < nch)
                def _():
                    eload(jj + 1, b1, True)
                    gather(jj + 1, b1, False)

                @pl.when(jj >= 1)
                def _():
                    scat_wait(b2)

                @pl.when(jj + 2 < nch)
                def _():
                    eload(jj + 2, b2, False)

                gather(jj, b0, True)

                def group(g, cc):
                    wg = wv[b0, 0, pl.ds(g * 16, 16)]
                    for i in range(16):
                        ww = wg[i]
                        e = g * 16 + i
                        for t in range(D // 16):
                            sl = pl.ds(t * 16, 16)
                            rows3[b0, e, sl] = rows3[b0, e, sl] * ww
                    return cc

                lax.fori_loop(0, C // 16, group, 0)
                pltpu.async_copy(rows3.at[b0], acc.at[div.at[b0, 0]], ssem, add=True)
            return carry

        lax.fori_loop(0, nch // 3, outer, 0)
        scat_wait(2)
        plsc.subcore_barrier()
        pltpu.sync_copy(acc.at[pl.ds(r0, RPT)], out_hbm.at[c, pl.ds(r0, RPT)])

    return k(h, srcp, dstp, wp)


def _combine_body(p_ref, o_ref):
    o_ref[...] = p_ref[0] + p_ref[1]


def _combine(p):
    blk = 2048
    return pl.pallas_call(
        _combine_body,
        grid=(NP // blk,),
        in_specs=[pl.BlockSpec((2, blk, D), lambda i: (0, i, 0))],
        out_specs=pl.BlockSpec((blk, D), lambda i: (i, 0)),
        out_shape=jax.ShapeDtypeStruct((NP, D), jnp.float32),
    )(p)


def _proj_body(h_ref, f1_ref, p2_ref, wt_ref, b_ref, o_ref, *, act):
    f2 = p2_ref[0] + p2_ref[1]
    z = (
        jnp.dot(h_ref[...], wt_ref[0:D], preferred_element_type=jnp.float32)
        + jnp.dot(f1_ref[...], wt_ref[D:2 * D], preferred_element_type=jnp.float32)
        + jnp.dot(f2, wt_ref[2 * D:3 * D], preferred_element_type=jnp.float32)
        + b_ref[...]
    )
    if act:
        z = jnp.where(z >= 0, z, 0.01 * z)
    o_ref[...] = z


def _proj(h, f1, p2, W, b, act, out_rows, blk):
    wt = W.T
    out_d = W.shape[0]
    rspec = pl.BlockSpec((blk, D), lambda i: (i, 0))
    return pl.pallas_call(
        functools.partial(_proj_body, act=act),
        grid=(out_rows // blk,),
        in_specs=[
            rspec, rspec,
            pl.BlockSpec((2, blk, D), lambda i: (0, i, 0)),
            pl.BlockSpec((3 * D, out_d), lambda i: (0, 0)),
            pl.BlockSpec((out_d,), lambda i: (0,)),
        ],
        out_specs=pl.BlockSpec((blk, out_d), lambda i: (i, 0)),
        out_shape=jax.ShapeDtypeStruct((out_rows, out_d), jnp.float32),
    )(h, f1, p2, wt, b)


def kernel(x, edge_index, edge_weight, W0, b0, W1, b1):
    dst = edge_index[0]
    src = edge_index[1]
    pad = EPTOT - E
    srcp = jnp.pad(src.astype(jnp.int32), (0, pad))
    dstp = jnp.pad(dst.astype(jnp.int32), (0, pad)).reshape(16 * NCHT, 1, C)
    wp = jnp.pad(edge_weight, (0, pad))

    f1 = _combine(_sc_spmm(x, srcp, dstp, wp))
    p2 = _sc_spmm(f1, srcp, dstp, wp)
    h1 = _proj(x, f1, p2, W0, b0, True, NP, 2048)
    g1 = _combine(_sc_spmm(h1, srcp, dstp, wp))
    pg2 = _sc_spmm(g1, srcp, dstp, wp)
    return _proj(h1, g1, pg2, W1, b1, False, N, 1000)
